# baseline (speedup 1.0000x reference)
import jax
import jax.numpy as jnp
from jax.experimental import pallas as pl

_P_SA0 = 0.1
_P_SA1 = 0.1
_G_SA0 = 0.003
_G_SA1 = 3e-06


def _saf_body(x_ref, p_ref, o_ref):
    p = p_ref[...]
    out = jnp.where(p <= jnp.float32(_P_SA0), jnp.float32(_G_SA0), x_ref[...])
    o_ref[...] = jnp.where(p > jnp.float32(1.0 - _P_SA1), jnp.float32(_G_SA1), out)


def kernel(input, p_state):
    M, N = input.shape
    BM = 256
    grid = (M // BM,)
    return pl.pallas_call(
        _saf_body,
        grid=grid,
        in_specs=[
            pl.BlockSpec((BM, N), lambda i: (i, 0)),
            pl.BlockSpec((BM, N), lambda i: (i, 0)),
        ],
        out_specs=pl.BlockSpec((BM, N), lambda i: (i, 0)),
        out_shape=jax.ShapeDtypeStruct((M, N), jnp.float32),
    )(input, p_state)


# speedup vs baseline: 1.0023x; 1.0023x over previous
import jax
import jax.numpy as jnp
from jax.experimental import pallas as pl

_P_SA0 = 0.1
_P_SA1 = 0.1
_G_SA0 = 0.003
_G_SA1 = 3e-06


def _saf_body(x_ref, p_ref, o_ref):
    p = p_ref[...]
    out = jnp.where(p <= jnp.float32(_P_SA0), jnp.float32(_G_SA0), x_ref[...])
    o_ref[...] = jnp.where(p > jnp.float32(1.0 - _P_SA1), jnp.float32(_G_SA1), out)


def kernel(input, p_state):
    M, N = input.shape
    BM = 512
    grid = (M // BM,)
    return pl.pallas_call(
        _saf_body,
        grid=grid,
        in_specs=[
            pl.BlockSpec((BM, N), lambda i: (i, 0)),
            pl.BlockSpec((BM, N), lambda i: (i, 0)),
        ],
        out_specs=pl.BlockSpec((BM, N), lambda i: (i, 0)),
        out_shape=jax.ShapeDtypeStruct((M, N), jnp.float32),
    )(input, p_state)
